# baseline (speedup 1.0000x reference)
import functools

import jax
import jax.numpy as jnp
from jax import lax
from jax.experimental import pallas as pl
from jax.experimental.pallas import tpu as pltpu
from jax.experimental.pallas import tpu_sc as plsc

_C = 6625
_D = 512
_B = 64
_S = 80
_N = _B * _S
_CC = 1664
_NCC = (_C + _CC - 1) // _CC
_NEG = -3.4e38


def _prep_body(label_len_ref, predicts_ref, labels_ref, labs_ref, w_ref,
               rm_ref, ri_ref):
    b = pl.program_id(0)
    c = pl.program_id(1)

    @pl.when(c == 0)
    def _():
        rm_ref[...] = jnp.full((_S, 1), _NEG, jnp.float32)
        ri_ref[...] = jnp.zeros((_S, 1), jnp.int32)

    x = predicts_ref[...]
    gi = lax.broadcasted_iota(jnp.int32, (_S, _CC), 1) + c * _CC
    xm = jnp.where(gi < _C, x, _NEG)
    m_c = jnp.max(xm, axis=1, keepdims=True)
    i_c = jnp.min(jnp.where(xm == m_c, gi, _C), axis=1, keepdims=True)

    rm = rm_ref[...]
    ri = ri_ref[...]
    upd = m_c > rm
    rm_ref[...] = jnp.where(upd, m_c, rm)
    ri_ref[...] = jnp.where(upd, i_c, ri)

    @pl.when(c == _NCC - 1)
    def _():
        raw = ri_ref[...]
        prev = jnp.concatenate(
            [jnp.full((1, 1), -1, jnp.int32), raw[:-1]], axis=0)
        char_rep = prev == raw
        is_char = raw > 0
        mk = jnp.logical_and(
            is_char, jnp.logical_not(char_rep)).astype(jnp.float32)

        count = jnp.sum(mk)
        valid = (count == label_len_ref[b].astype(jnp.float32)
                 ).astype(jnp.float32)

        ri2 = lax.broadcasted_iota(jnp.int32, (_S, _S), 0)
        ti = lax.broadcasted_iota(jnp.int32, (_S, _S), 1)
        ltri = (ti <= ri2).astype(jnp.float32)
        cs = jnp.dot(ltri, mk, preferred_element_type=jnp.float32)
        rank = jnp.clip(cs.astype(jnp.int32) - 1, 0, _S - 1)

        onehot = (rank == ti).astype(jnp.float32)
        labels_col = labels_ref[...].astype(jnp.float32)
        labs_f = jnp.dot(onehot, labels_col,
                         preferred_element_type=jnp.float32)

        labs_ref[...] = labs_f.astype(jnp.int32)
        w_ref[...] = jnp.broadcast_to(mk * valid, (_S, 16))


def _prep(predicts2, labels2, label_len):
    return pl.pallas_call(
        _prep_body,
        grid=(_B, _NCC),
        in_specs=[
            pl.BlockSpec(memory_space=pltpu.SMEM),
            pl.BlockSpec((_S, _CC), lambda b, c: (b, c)),
            pl.BlockSpec((_S, 1), lambda b, c: (b, 0)),
        ],
        out_specs=[
            pl.BlockSpec((_S, 1), lambda b, c: (b, 0)),
            pl.BlockSpec((_S, 16), lambda b, c: (b, 0)),
        ],
        out_shape=[
            jax.ShapeDtypeStruct((_N, 1), jnp.int32),
            jax.ShapeDtypeStruct((_N, 16), jnp.float32),
        ],
        scratch_shapes=[
            pltpu.VMEM((_S, 1), jnp.float32),
            pltpu.VMEM((_S, 1), jnp.int32),
        ],
    )(label_len, predicts2, labels2)


_NC, _NS = 2, 16
_NW = _NC * _NS
_PER_W = _N // _NW
_CHUNK = 40
_NCHUNK = _PER_W // _CHUNK


def _sc_body(centers_hbm, labs_hbm, w_hbm, emb_hbm, out_sq_hbm, out_w_hbm,
             idx_v, w_v, c_v0, c_v1, e_v0, e_v1, res_v,
             sem_c0, sem_c1, sem_e0, sem_e1):
    wid = lax.axis_index("s") * _NC + lax.axis_index("c")
    base = wid * _PER_W
    pltpu.sync_copy(labs_hbm.at[pl.ds(base, _PER_W)], idx_v)
    pltpu.sync_copy(w_hbm.at[pl.ds(base, _PER_W)], w_v)

    cbuf = (c_v0, c_v1)
    ebuf = (e_v0, e_v1)
    csem = (sem_c0, sem_c1)
    esem = (sem_e0, sem_e1)

    def start(g):
        slot = g % 2
        dc = pltpu.async_copy(
            centers_hbm.at[idx_v.at[pl.ds(g * _CHUNK, _CHUNK)]],
            cbuf[slot], csem[slot])
        de = pltpu.async_copy(
            emb_hbm.at[pl.ds(base + g * _CHUNK, _CHUNK)],
            ebuf[slot], esem[slot])
        return dc, de

    pend = start(0)
    acc = jnp.zeros((16,), jnp.float32)
    wacc = jnp.zeros((16,), jnp.float32)
    for g in range(_NCHUNK):
        nxt = start(g + 1) if g + 1 < _NCHUNK else None
        pend[0].wait()
        pend[1].wait()
        c_v = cbuf[g % 2]
        e_v = ebuf[g % 2]

        def row_body(r, carry):
            acc, wacc = carry
            wspl = w_v[g * _CHUNK + r, :]
            s = jnp.zeros((16,), jnp.float32)
            for k in range(_D // 16):
                ev = e_v[r, pl.ds(k * 16, 16)]
                cv = c_v[r, pl.ds(k * 16, 16)]
                d = ev - cv
                s = s + d * d
            return acc + wspl * s, wacc + wspl

        acc, wacc = lax.fori_loop(0, _CHUNK, row_body, (acc, wacc))
        pend = nxt

    res_v[0, :] = acc
    res_v[1, :] = wacc
    pltpu.sync_copy(res_v.at[0], out_sq_hbm.at[wid])
    pltpu.sync_copy(res_v.at[1], out_w_hbm.at[wid])


def _sc_loss(centers, labs_flat, w16, emb_flat):
    mesh = plsc.VectorSubcoreMesh(
        core_axis_name="c", subcore_axis_name="s")
    run = pl.kernel(
        _sc_body,
        out_type=[
            jax.ShapeDtypeStruct((_NW, 16), jnp.float32),
            jax.ShapeDtypeStruct((_NW, 16), jnp.float32),
        ],
        mesh=mesh,
        scratch_types=[
            pltpu.VMEM((_PER_W,), jnp.int32),
            pltpu.VMEM((_PER_W, 16), jnp.float32),
            pltpu.VMEM((_CHUNK, _D), jnp.float32),
            pltpu.VMEM((_CHUNK, _D), jnp.float32),
            pltpu.VMEM((_CHUNK, _D), jnp.float32),
            pltpu.VMEM((_CHUNK, _D), jnp.float32),
            pltpu.VMEM((2, 16), jnp.float32),
            pltpu.SemaphoreType.DMA,
            pltpu.SemaphoreType.DMA,
            pltpu.SemaphoreType.DMA,
            pltpu.SemaphoreType.DMA,
        ],
    )
    return run(centers, labs_flat, w16, emb_flat)


@jax.jit
def kernel(predicts, embedding, labels, label_len, centers):
    predicts2 = predicts.reshape(_N, _C)
    labels2 = labels.reshape(_N, 1)
    labs, w16 = _prep(predicts2, labels2, label_len)
    labs_flat = labs.reshape(_N)
    emb_flat = embedding.reshape(_N, _D)
    part_sq, part_w = _sc_loss(centers, labs_flat, w16, emb_flat)
    total = jnp.sum(part_sq)
    wsum = jnp.sum(part_w) / 16.0
    return total / (wsum * _D)


# speedup vs baseline: 1.9808x; 1.9808x over previous
import functools

import jax
import jax.numpy as jnp
from jax import lax
from jax.experimental import pallas as pl
from jax.experimental.pallas import tpu as pltpu
from jax.experimental.pallas import tpu_sc as plsc

_C = 6625
_D = 512
_B = 64
_S = 80
_N = _B * _S
_CC = 1664
_NCC = (_C + _CC - 1) // _CC
_NEG = -3.4e38


def _prep_body(label_len_ref, p0, p1, p2, p3, labels_ref, labs_ref, w_ref):
    b = pl.program_id(0)

    ms, js = [], []
    for k, pref in enumerate((p0, p1, p2, p3)):
        x = pref[...]
        gi = lax.broadcasted_iota(jnp.int32, (_S, _CC), 1) + k * _CC
        if (k + 1) * _CC > _C:
            x = jnp.where(gi < _C, x, _NEG)
        m_k = jnp.max(x, axis=1, keepdims=True)
        i_k = jnp.min(jnp.where(x == m_k, gi, _C), axis=1, keepdims=True)
        ms.append(m_k)
        js.append(i_k)

    def merge(a, b):
        (ma, ia), (mb, ib) = a, b
        take_b = mb > ma
        return jnp.where(take_b, mb, ma), jnp.where(take_b, ib, ia)

    _, raw = merge(merge((ms[0], js[0]), (ms[1], js[1])),
                   merge((ms[2], js[2]), (ms[3], js[3])))

    prev = jnp.concatenate(
        [jnp.full((1, 1), -1, jnp.int32), raw[:-1]], axis=0)
    char_rep = prev == raw
    is_char = raw > 0
    mk = jnp.logical_and(
        is_char, jnp.logical_not(char_rep)).astype(jnp.float32)

    count = jnp.sum(mk)
    valid = (count == label_len_ref[b].astype(jnp.float32)
             ).astype(jnp.float32)

    ri2 = lax.broadcasted_iota(jnp.int32, (_S, _S), 0)
    ti = lax.broadcasted_iota(jnp.int32, (_S, _S), 1)
    ltri = (ti <= ri2).astype(jnp.float32)
    cs = jnp.dot(ltri, mk, preferred_element_type=jnp.float32)
    rank = jnp.clip(cs.astype(jnp.int32) - 1, 0, _S - 1)

    onehot = (rank == ti).astype(jnp.float32)
    labels_col = labels_ref[...].astype(jnp.float32)
    labs_f = jnp.dot(onehot, labels_col,
                     preferred_element_type=jnp.float32)

    labs_ref[...] = labs_f.astype(jnp.int32)
    w_ref[...] = jnp.broadcast_to(mk * valid, (_S, 16))


def _prep(predicts2, labels2, label_len):
    pspec = [pl.BlockSpec((_S, _CC), (lambda b, _k=k: (b, _k)))
             for k in range(_NCC)]
    return pl.pallas_call(
        _prep_body,
        grid=(_B,),
        in_specs=[
            pl.BlockSpec(memory_space=pltpu.SMEM),
            *pspec,
            pl.BlockSpec((_S, 1), lambda b: (b, 0)),
        ],
        out_specs=[
            pl.BlockSpec((_S, 1), lambda b: (b, 0)),
            pl.BlockSpec((_S, 16), lambda b: (b, 0)),
        ],
        out_shape=[
            jax.ShapeDtypeStruct((_N, 1), jnp.int32),
            jax.ShapeDtypeStruct((_N, 16), jnp.float32),
        ],
    )(label_len, predicts2, predicts2, predicts2, predicts2, labels2)


_NC, _NS = 2, 16
_NW = _NC * _NS
_PER_W = _N // _NW
_CHUNK = 40
_NCHUNK = _PER_W // _CHUNK


def _sc_body(centers_hbm, labs_hbm, w_hbm, emb_hbm, out_sq_hbm, out_w_hbm,
             idx_v, w_v, c_v0, c_v1, e_v0, e_v1, res_v,
             sem_c0, sem_c1, sem_e0, sem_e1):
    wid = lax.axis_index("s") * _NC + lax.axis_index("c")
    base = wid * _PER_W
    pltpu.sync_copy(labs_hbm.at[pl.ds(base, _PER_W)], idx_v)
    pltpu.sync_copy(w_hbm.at[pl.ds(base, _PER_W)], w_v)

    cbuf = (c_v0, c_v1)
    ebuf = (e_v0, e_v1)
    csem = (sem_c0, sem_c1)
    esem = (sem_e0, sem_e1)

    def start(g):
        slot = g % 2
        dc = pltpu.async_copy(
            centers_hbm.at[idx_v.at[pl.ds(g * _CHUNK, _CHUNK)]],
            cbuf[slot], csem[slot])
        de = pltpu.async_copy(
            emb_hbm.at[pl.ds(base + g * _CHUNK, _CHUNK)],
            ebuf[slot], esem[slot])
        return dc, de

    pend = start(0)
    acc = jnp.zeros((16,), jnp.float32)
    wacc = jnp.zeros((16,), jnp.float32)
    for g in range(_NCHUNK):
        nxt = start(g + 1) if g + 1 < _NCHUNK else None
        pend[0].wait()
        pend[1].wait()
        c_v = cbuf[g % 2]
        e_v = ebuf[g % 2]

        def row_body(r, carry):
            acc, wacc = carry
            wspl = w_v[g * _CHUNK + r, :]
            s = jnp.zeros((16,), jnp.float32)
            for k in range(_D // 16):
                ev = e_v[r, pl.ds(k * 16, 16)]
                cv = c_v[r, pl.ds(k * 16, 16)]
                d = ev - cv
                s = s + d * d
            return acc + wspl * s, wacc + wspl

        acc, wacc = lax.fori_loop(0, _CHUNK, row_body, (acc, wacc))
        pend = nxt

    res_v[0, :] = acc
    res_v[1, :] = wacc
    pltpu.sync_copy(res_v.at[0], out_sq_hbm.at[wid])
    pltpu.sync_copy(res_v.at[1], out_w_hbm.at[wid])


def _sc_loss(centers, labs_flat, w16, emb_flat):
    mesh = plsc.VectorSubcoreMesh(
        core_axis_name="c", subcore_axis_name="s")
    run = pl.kernel(
        _sc_body,
        out_type=[
            jax.ShapeDtypeStruct((_NW, 16), jnp.float32),
            jax.ShapeDtypeStruct((_NW, 16), jnp.float32),
        ],
        mesh=mesh,
        scratch_types=[
            pltpu.VMEM((_PER_W,), jnp.int32),
            pltpu.VMEM((_PER_W, 16), jnp.float32),
            pltpu.VMEM((_CHUNK, _D), jnp.float32),
            pltpu.VMEM((_CHUNK, _D), jnp.float32),
            pltpu.VMEM((_CHUNK, _D), jnp.float32),
            pltpu.VMEM((_CHUNK, _D), jnp.float32),
            pltpu.VMEM((2, 16), jnp.float32),
            pltpu.SemaphoreType.DMA,
            pltpu.SemaphoreType.DMA,
            pltpu.SemaphoreType.DMA,
            pltpu.SemaphoreType.DMA,
        ],
    )
    return run(centers, labs_flat, w16, emb_flat)


@jax.jit
def kernel(predicts, embedding, labels, label_len, centers):
    predicts2 = predicts.reshape(_N, _C)
    labels2 = labels.reshape(_N, 1)
    labs, w16 = _prep(predicts2, labels2, label_len)
    labs_flat = labs.reshape(_N)
    emb_flat = embedding.reshape(_N, _D)
    part_sq, part_w = _sc_loss(centers, labs_flat, w16, emb_flat)
    total = jnp.sum(part_sq)
    wsum = jnp.sum(part_w) / 16.0
    return total / (wsum * _D)


# speedup vs baseline: 2.3464x; 1.1846x over previous
import functools

import jax
import jax.numpy as jnp
from jax import lax
from jax.experimental import pallas as pl
from jax.experimental.pallas import tpu as pltpu
from jax.experimental.pallas import tpu_sc as plsc

_C = 6625
_D = 512
_B = 64
_S = 80
_N = _B * _S
_NEG = -3.4e38


_R1A = 320
_G1A = _N // _R1A


def _argmax_body(p_ref, raw_ref):
    x = p_ref[...]
    m = jnp.max(x, axis=1, keepdims=True)
    ci = lax.broadcasted_iota(jnp.int32, (_R1A, _C), 1)
    raw_ref[...] = jnp.min(
        jnp.where(x == m, ci, _C), axis=1, keepdims=True)


def _argmax(predicts2):
    return pl.pallas_call(
        _argmax_body,
        grid=(_G1A,),
        in_specs=[pl.BlockSpec((_R1A, _C), lambda r: (r, 0))],
        out_specs=pl.BlockSpec((_R1A, 1), lambda r: (r, 0)),
        out_shape=jax.ShapeDtypeStruct((_N, 1), jnp.int32),
    )(predicts2)


_R1B = 320
_G1B = _N // _R1B


def _ctc_body(raw_ref, labels_ref, ll_ref, labs_ref, w_ref):
    raw = raw_ref[...]
    ri = lax.broadcasted_iota(jnp.int32, (_R1B, 1), 0)
    prev = jnp.concatenate(
        [jnp.full((1, 1), -1, jnp.int32), raw[:-1]], axis=0)
    first = (ri % _S) == 0
    char_rep = jnp.logical_and(prev == raw, jnp.logical_not(first))
    mk = jnp.logical_and(raw > 0, jnp.logical_not(char_rep)
                         ).astype(jnp.float32)

    rr = lax.broadcasted_iota(jnp.int32, (_R1B, _R1B), 0)
    tt = lax.broadcasted_iota(jnp.int32, (_R1B, _R1B), 1)
    sameseg = (rr // _S) == (tt // _S)
    seg_f = sameseg.astype(jnp.float32)
    ltri = jnp.where(tt <= rr, seg_f, 0.0)

    cs = jnp.dot(ltri, mk, preferred_element_type=jnp.float32)
    cnt = jnp.dot(seg_f, mk, preferred_element_type=jnp.float32)
    valid = (cnt == ll_ref[...]).astype(jnp.float32)
    rank = jnp.clip(cs.astype(jnp.int32) - 1, 0, _S - 1)

    tloc = tt % _S
    onehot = jnp.where(tloc == rank, seg_f, 0.0)
    labels_col = labels_ref[...].astype(jnp.float32)
    labs_f = jnp.dot(onehot, labels_col,
                     preferred_element_type=jnp.float32)

    labs_ref[...] = labs_f.astype(jnp.int32)
    w_ref[...] = jnp.broadcast_to(mk * valid, (_R1B, 16))


def _ctc(raw, labels2, ll_rep):
    return pl.pallas_call(
        _ctc_body,
        grid=(_G1B,),
        in_specs=[
            pl.BlockSpec((_R1B, 1), lambda r: (r, 0)),
            pl.BlockSpec((_R1B, 1), lambda r: (r, 0)),
            pl.BlockSpec((_R1B, 1), lambda r: (r, 0)),
        ],
        out_specs=[
            pl.BlockSpec((_R1B, 1), lambda r: (r, 0)),
            pl.BlockSpec((_R1B, 16), lambda r: (r, 0)),
        ],
        out_shape=[
            jax.ShapeDtypeStruct((_N, 1), jnp.int32),
            jax.ShapeDtypeStruct((_N, 16), jnp.float32),
        ],
    )(raw, labels2, ll_rep)


_NC, _NS = 2, 16
_NW = _NC * _NS
_PER_W = _N // _NW
_CHUNK = 40
_NCHUNK = _PER_W // _CHUNK


def _sc_body(centers_hbm, labs_hbm, w_hbm, emb_hbm, out_sq_hbm, out_w_hbm,
             idx_v, w_v, c_v0, c_v1, e_v0, e_v1, res_v,
             sem_c0, sem_c1, sem_e0, sem_e1):
    wid = lax.axis_index("s") * _NC + lax.axis_index("c")
    base = wid * _PER_W
    pltpu.sync_copy(labs_hbm.at[pl.ds(base, _PER_W)], idx_v)
    pltpu.sync_copy(w_hbm.at[pl.ds(base, _PER_W)], w_v)

    cbuf = (c_v0, c_v1)
    ebuf = (e_v0, e_v1)
    csem = (sem_c0, sem_c1)
    esem = (sem_e0, sem_e1)

    def start(g):
        slot = g % 2
        dc = pltpu.async_copy(
            centers_hbm.at[idx_v.at[pl.ds(g * _CHUNK, _CHUNK)]],
            cbuf[slot], csem[slot])
        de = pltpu.async_copy(
            emb_hbm.at[pl.ds(base + g * _CHUNK, _CHUNK)],
            ebuf[slot], esem[slot])
        return dc, de

    pend = start(0)
    acc = jnp.zeros((16,), jnp.float32)
    wacc = jnp.zeros((16,), jnp.float32)
    for g in range(_NCHUNK):
        nxt = start(g + 1) if g + 1 < _NCHUNK else None
        pend[0].wait()
        pend[1].wait()
        c_v = cbuf[g % 2]
        e_v = ebuf[g % 2]

        def row_body(r, carry):
            acc, wacc = carry
            wspl = w_v[g * _CHUNK + r, :]
            s = jnp.zeros((16,), jnp.float32)
            for k in range(_D // 16):
                ev = e_v[r, pl.ds(k * 16, 16)]
                cv = c_v[r, pl.ds(k * 16, 16)]
                d = ev - cv
                s = s + d * d
            return acc + wspl * s, wacc + wspl

        acc, wacc = lax.fori_loop(0, _CHUNK, row_body, (acc, wacc))
        pend = nxt

    res_v[0, :] = acc
    res_v[1, :] = wacc
    pltpu.sync_copy(res_v.at[0], out_sq_hbm.at[wid])
    pltpu.sync_copy(res_v.at[1], out_w_hbm.at[wid])


def _sc_loss(centers, labs_flat, w16, emb_flat):
    mesh = plsc.VectorSubcoreMesh(
        core_axis_name="c", subcore_axis_name="s")
    run = pl.kernel(
        _sc_body,
        out_type=[
            jax.ShapeDtypeStruct((_NW, 16), jnp.float32),
            jax.ShapeDtypeStruct((_NW, 16), jnp.float32),
        ],
        mesh=mesh,
        scratch_types=[
            pltpu.VMEM((_PER_W,), jnp.int32),
            pltpu.VMEM((_PER_W, 16), jnp.float32),
            pltpu.VMEM((_CHUNK, _D), jnp.float32),
            pltpu.VMEM((_CHUNK, _D), jnp.float32),
            pltpu.VMEM((_CHUNK, _D), jnp.float32),
            pltpu.VMEM((_CHUNK, _D), jnp.float32),
            pltpu.VMEM((2, 16), jnp.float32),
            pltpu.SemaphoreType.DMA,
            pltpu.SemaphoreType.DMA,
            pltpu.SemaphoreType.DMA,
            pltpu.SemaphoreType.DMA,
        ],
    )
    return run(centers, labs_flat, w16, emb_flat)


@jax.jit
def kernel(predicts, embedding, labels, label_len, centers):
    predicts2 = predicts.reshape(_N, _C)
    labels2 = labels.reshape(_N, 1)
    ll_rep = jnp.broadcast_to(
        label_len.astype(jnp.float32)[:, None], (_B, _S)).reshape(_N, 1)
    raw = _argmax(predicts2)
    labs, w16 = _ctc(raw, labels2, ll_rep)
    labs_flat = labs.reshape(_N)
    emb_flat = embedding.reshape(_N, _D)
    part_sq, part_w = _sc_loss(centers, labs_flat, w16, emb_flat)
    total = jnp.sum(part_sq)
    wsum = jnp.sum(part_w) / 16.0
    return total / (wsum * _D)


# speedup vs baseline: 2.6220x; 1.1175x over previous
import functools

import jax
import jax.numpy as jnp
from jax import lax
from jax.experimental import pallas as pl
from jax.experimental.pallas import tpu as pltpu
from jax.experimental.pallas import tpu_sc as plsc

_C = 6625
_D = 512
_B = 64
_S = 80
_N = _B * _S
_NEG = -3.4e38


_R1A = 320
_G1A = _N // _R1A


_R1B = _R1A


def _prep_body(p_ref, labels_ref, ll_ref, labs_ref, w_ref):
    x = p_ref[...]
    m = jnp.max(x, axis=1, keepdims=True)
    ci = lax.broadcasted_iota(jnp.int32, (_R1A, _C), 1)
    raw = jnp.min(jnp.where(x == m, ci, _C), axis=1, keepdims=True)

    ri = lax.broadcasted_iota(jnp.int32, (_R1B, 1), 0)
    prev = jnp.concatenate(
        [jnp.full((1, 1), -1, jnp.int32), raw[:-1]], axis=0)
    first = (ri % _S) == 0
    char_rep = jnp.logical_and(prev == raw, jnp.logical_not(first))
    mk = jnp.logical_and(raw > 0, jnp.logical_not(char_rep)
                         ).astype(jnp.float32)

    rr = lax.broadcasted_iota(jnp.int32, (_R1B, _R1B), 0)
    tt = lax.broadcasted_iota(jnp.int32, (_R1B, _R1B), 1)
    sameseg = (rr // _S) == (tt // _S)
    seg_f = sameseg.astype(jnp.float32)
    ltri = jnp.where(tt <= rr, seg_f, 0.0)

    cs = jnp.dot(ltri, mk, preferred_element_type=jnp.float32)
    cnt = jnp.dot(seg_f, mk, preferred_element_type=jnp.float32)
    valid = (cnt == ll_ref[...]).astype(jnp.float32)
    rank = jnp.clip(cs.astype(jnp.int32) - 1, 0, _S - 1)

    tloc = tt % _S
    onehot = jnp.where(tloc == rank, seg_f, 0.0)
    labels_col = labels_ref[...].astype(jnp.float32)
    labs_f = jnp.dot(onehot, labels_col,
                     preferred_element_type=jnp.float32)

    labs_ref[...] = labs_f.astype(jnp.int32)
    w_ref[...] = jnp.broadcast_to(mk * valid, (_R1B, 16))


def _prep(predicts2, labels2, ll_rep):
    return pl.pallas_call(
        _prep_body,
        grid=(_G1A,),
        in_specs=[
            pl.BlockSpec((_R1A, _C), lambda r: (r, 0)),
            pl.BlockSpec((_R1B, 1), lambda r: (r, 0)),
            pl.BlockSpec((_R1B, 1), lambda r: (r, 0)),
        ],
        out_specs=[
            pl.BlockSpec((_R1B, 1), lambda r: (r, 0)),
            pl.BlockSpec((_R1B, 16), lambda r: (r, 0)),
        ],
        out_shape=[
            jax.ShapeDtypeStruct((_N, 1), jnp.int32),
            jax.ShapeDtypeStruct((_N, 16), jnp.float32),
        ],
    )(predicts2, labels2, ll_rep)


_NC, _NS = 2, 16
_NW = _NC * _NS
_PER_W = _N // _NW
_CHUNK = 40
_NCHUNK = _PER_W // _CHUNK


def _sc_body(centers_hbm, labs_hbm, w_hbm, emb_hbm, out_sq_hbm, out_w_hbm,
             idx_v, w_v, c_v0, c_v1, e_v0, e_v1, res_v,
             sem_c0, sem_c1, sem_e0, sem_e1):
    wid = lax.axis_index("s") * _NC + lax.axis_index("c")
    base = wid * _PER_W
    pltpu.sync_copy(labs_hbm.at[pl.ds(base, _PER_W)], idx_v)
    pltpu.sync_copy(w_hbm.at[pl.ds(base, _PER_W)], w_v)

    cbuf = (c_v0, c_v1)
    ebuf = (e_v0, e_v1)
    csem = (sem_c0, sem_c1)
    esem = (sem_e0, sem_e1)

    def start(g):
        slot = g % 2
        dc = pltpu.async_copy(
            centers_hbm.at[idx_v.at[pl.ds(g * _CHUNK, _CHUNK)]],
            cbuf[slot], csem[slot])
        de = pltpu.async_copy(
            emb_hbm.at[pl.ds(base + g * _CHUNK, _CHUNK)],
            ebuf[slot], esem[slot])
        return dc, de

    pend = start(0)
    acc = jnp.zeros((16,), jnp.float32)
    wacc = jnp.zeros((16,), jnp.float32)
    for g in range(_NCHUNK):
        nxt = start(g + 1) if g + 1 < _NCHUNK else None
        pend[0].wait()
        pend[1].wait()
        c_v = cbuf[g % 2]
        e_v = ebuf[g % 2]

        def row_body(r, carry):
            acc, wacc = carry
            wspl = w_v[g * _CHUNK + r, :]
            s = jnp.zeros((16,), jnp.float32)
            for k in range(_D // 16):
                ev = e_v[r, pl.ds(k * 16, 16)]
                cv = c_v[r, pl.ds(k * 16, 16)]
                d = ev - cv
                s = s + d * d
            return acc + wspl * s, wacc + wspl

        acc, wacc = lax.fori_loop(0, _CHUNK, row_body, (acc, wacc))
        pend = nxt

    res_v[0, :] = acc
    res_v[1, :] = wacc
    pltpu.sync_copy(res_v.at[0], out_sq_hbm.at[wid])
    pltpu.sync_copy(res_v.at[1], out_w_hbm.at[wid])


def _sc_loss(centers, labs_flat, w16, emb_flat):
    mesh = plsc.VectorSubcoreMesh(
        core_axis_name="c", subcore_axis_name="s")
    run = pl.kernel(
        _sc_body,
        out_type=[
            jax.ShapeDtypeStruct((_NW, 16), jnp.float32),
            jax.ShapeDtypeStruct((_NW, 16), jnp.float32),
        ],
        mesh=mesh,
        scratch_types=[
            pltpu.VMEM((_PER_W,), jnp.int32),
            pltpu.VMEM((_PER_W, 16), jnp.float32),
            pltpu.VMEM((_CHUNK, _D), jnp.float32),
            pltpu.VMEM((_CHUNK, _D), jnp.float32),
            pltpu.VMEM((_CHUNK, _D), jnp.float32),
            pltpu.VMEM((_CHUNK, _D), jnp.float32),
            pltpu.VMEM((2, 16), jnp.float32),
            pltpu.SemaphoreType.DMA,
            pltpu.SemaphoreType.DMA,
            pltpu.SemaphoreType.DMA,
            pltpu.SemaphoreType.DMA,
        ],
    )
    return run(centers, labs_flat, w16, emb_flat)


@jax.jit
def kernel(predicts, embedding, labels, label_len, centers):
    predicts2 = predicts.reshape(_N, _C)
    labels2 = labels.reshape(_N, 1)
    ll_rep = jnp.broadcast_to(
        label_len.astype(jnp.float32)[:, None], (_B, _S)).reshape(_N, 1)
    labs, w16 = _prep(predicts2, labels2, ll_rep)
    labs_flat = labs.reshape(_N)
    emb_flat = embedding.reshape(_N, _D)
    part_sq, part_w = _sc_loss(centers, labs_flat, w16, emb_flat)
    total = jnp.sum(part_sq)
    wsum = jnp.sum(part_w) / 16.0
    return total / (wsum * _D)
